# baseline (device time: 59093 ns/iter reference)
import jax
import jax.numpy as jnp
from jax import lax
from jax.experimental import pallas as pl
from jax.experimental.pallas import tpu as pltpu

N_DEV = 4
B = 2
S_LOC = 128
S = S_LOC * N_DEV
D = 512
H_LOC = 8
DH = 64


def kernel(x, Wq, Wo, Wk, Wv):
    def body(x_ref, wq_ref, wo_ref, wk_ref, wv_ref, out_ref,
             xfull, ag_comm, osc, y_ref, rs_comm,
             ag_send, ag_recv, rs_send, rs_recv):
        my = lax.axis_index("i")
        left = lax.rem(my + N_DEV - 1, N_DEV)
        right = lax.rem(my + 1, N_DEV)

        barrier_sem = pltpu.get_barrier_semaphore()
        for nbr in (left, right):
            pl.semaphore_signal(
                barrier_sem, inc=1,
                device_id=(nbr,), device_id_type=pl.DeviceIdType.MESH,
            )
        pl.semaphore_wait(barrier_sem, 2)

        xbf = x_ref[...].astype(jnp.bfloat16)
        xfull[:, pl.ds(my * S_LOC, S_LOC), :] = xbf
        ag_comm[0] = xbf
        for h in range(N_DEV - 1):
            send_slot = h % 2
            recv_slot = (h + 1) % 2
            rdma = pltpu.make_async_remote_copy(
                src_ref=ag_comm.at[send_slot],
                dst_ref=ag_comm.at[recv_slot],
                send_sem=ag_send.at[send_slot],
                recv_sem=ag_recv.at[recv_slot],
                device_id=(right,),
                device_id_type=pl.DeviceIdType.MESH,
            )
            rdma.start()
            rdma.wait()
            origin = lax.rem(my - (h + 1) + N_DEV, N_DEV)
            xfull[:, pl.ds(origin * S_LOC, S_LOC), :] = ag_comm[recv_slot]

        wq = wq_ref[...].astype(jnp.bfloat16)
        wk = wk_ref[...].astype(jnp.bfloat16)
        wv = wv_ref[...].astype(jnp.bfloat16)
        wo = wo_ref[...].astype(jnp.bfloat16)
        for b in range(B):
            xb = xfull[b]
            qb = jnp.dot(xb, wq, preferred_element_type=jnp.float32)
            kb = jnp.dot(xb, wk, preferred_element_type=jnp.float32)
            vb = jnp.dot(xb, wv, preferred_element_type=jnp.float32).astype(
                jnp.bfloat16)
            qb = qb.astype(jnp.bfloat16)
            kb = kb.astype(jnp.bfloat16)
            for h in range(H_LOC):
                q = qb[:, h * DH:(h + 1) * DH]
                k = kb[:, h * DH:(h + 1) * DH]
                v = vb[:, h * DH:(h + 1) * DH]
                s = lax.dot_general(
                    q, k, (((1,), (1,)), ((), ())),
                    preferred_element_type=jnp.float32,
                ) * 0.125
                m = jnp.max(s, axis=1, keepdims=True)
                p = jnp.exp(s - m)
                l = jnp.sum(p, axis=1, keepdims=True)
                pn = (p / l).astype(jnp.bfloat16)
                o = jnp.dot(pn, v, preferred_element_type=jnp.float32)
                osc[b, :, h * DH:(h + 1) * DH] = o.astype(jnp.bfloat16)
            y_ref[b] = jnp.dot(osc[b], wo, preferred_element_type=jnp.float32)

        for s in range(N_DEV - 1):
            send_slot = s % 2
            recv_slot = (s + 1) % 2
            c_send = lax.rem(my - 1 - s + 2 * N_DEV, N_DEV)
            rs_comm[send_slot] = y_ref[:, pl.ds(c_send * S_LOC, S_LOC), :]
            rdma = pltpu.make_async_remote_copy(
                src_ref=rs_comm.at[send_slot],
                dst_ref=rs_comm.at[recv_slot],
                send_sem=rs_send.at[send_slot],
                recv_sem=rs_recv.at[recv_slot],
                device_id=(right,),
                device_id_type=pl.DeviceIdType.MESH,
            )
            rdma.start()
            rdma.wait()
            c_recv = lax.rem(my - 2 - s + 2 * N_DEV, N_DEV)
            acc = y_ref[:, pl.ds(c_recv * S_LOC, S_LOC), :] + rs_comm[recv_slot]
            y_ref[:, pl.ds(c_recv * S_LOC, S_LOC), :] = acc

        out_ref[...] = y_ref[:, pl.ds(my * S_LOC, S_LOC), :]

    return pl.pallas_call(
        body,
        out_shape=jax.ShapeDtypeStruct((B, S_LOC, D), jnp.float32),
        in_specs=[pl.BlockSpec(memory_space=pltpu.VMEM)] * 5,
        out_specs=pl.BlockSpec(memory_space=pltpu.VMEM),
        scratch_shapes=[
            pltpu.VMEM((B, S, D), jnp.bfloat16),
            pltpu.VMEM((2, B, S_LOC, D), jnp.bfloat16),
            pltpu.VMEM((B, S, D), jnp.bfloat16),
            pltpu.VMEM((B, S, D), jnp.float32),
            pltpu.VMEM((2, B, S_LOC, D), jnp.float32),
            pltpu.SemaphoreType.DMA((2,)),
            pltpu.SemaphoreType.DMA((2,)),
            pltpu.SemaphoreType.DMA((2,)),
            pltpu.SemaphoreType.DMA((2,)),
        ],
        compiler_params=pltpu.CompilerParams(collective_id=0),
    )(x, Wq, Wo, Wk, Wv)


# device time: 32503 ns/iter; 1.8181x vs baseline; 1.8181x over previous
import jax
import jax.numpy as jnp
from jax import lax
from jax.experimental import pallas as pl
from jax.experimental.pallas import tpu as pltpu

N_DEV = 4
B = 2
S_LOC = 128
S = S_LOC * N_DEV
D = 512
H_LOC = 8
DH = 64

BF = jnp.bfloat16
F32 = jnp.float32


def kernel(x, Wq, Wo, Wk, Wv):
    def body(x_ref, wq_ref, wo_ref, wk_ref, wv_ref, out_ref,
             xbf_buf, xg, qf, kf, vf, osc, y_ref, ysend, rsbuf,
             ag_send, ag_recv, rs_send, rs_recv):
        my = lax.axis_index("i")

        barrier_sem = pltpu.get_barrier_semaphore()
        for k in (1, 2, 3):
            pl.semaphore_signal(
                barrier_sem, inc=1,
                device_id=(lax.rem(my + k, N_DEV),),
                device_id_type=pl.DeviceIdType.MESH,
            )
        pl.semaphore_wait(barrier_sem, 3)

        xbf_buf[...] = x_ref[...].astype(BF)
        ag_rdmas = []
        for k in (1, 2, 3):
            r = 3 - k
            rdma = pltpu.make_async_remote_copy(
                src_ref=xbf_buf,
                dst_ref=xg.at[r],
                send_sem=ag_send.at[k - 1],
                recv_sem=ag_recv.at[r],
                device_id=(lax.rem(my + k, N_DEV),),
                device_id_type=pl.DeviceIdType.MESH,
            )
            rdma.start()
            ag_rdmas.append(rdma)

        wq = wq_ref[...].astype(BF)
        wk = wk_ref[...].astype(BF)
        wv = wv_ref[...].astype(BF)
        wo = wo_ref[...].astype(BF)

        def qkv_chunk(x_c, row0):
            for b in range(B):
                xcb = x_c[b]
                qf[b, pl.ds(row0, S_LOC), :] = jnp.dot(
                    xcb, wq, preferred_element_type=F32).astype(BF)
                kf[b, pl.ds(row0, S_LOC), :] = jnp.dot(
                    xcb, wk, preferred_element_type=F32).astype(BF)
                vf[b, pl.ds(row0, S_LOC), :] = jnp.dot(
                    xcb, wv, preferred_element_type=F32).astype(BF)

        qkv_chunk(xbf_buf[...], my * S_LOC)

        for r in (2, 0, 1):
            recv = pltpu.make_async_remote_copy(
                src_ref=xbf_buf,
                dst_ref=xg.at[r],
                send_sem=ag_send.at[0],
                recv_sem=ag_recv.at[r],
                device_id=(my,),
                device_id_type=pl.DeviceIdType.MESH,
            )
            recv.wait_recv()
            origin = lax.rem(my + r + 1, N_DEV)
            qkv_chunk(xg[r], origin * S_LOC)

        for b in range(B):
            qb = qf[b]
            kb = kf[b]
            vb = vf[b]
            for h in range(H_LOC):
                q = qb[:, h * DH:(h + 1) * DH]
                k_ = kb[:, h * DH:(h + 1) * DH]
                v = vb[:, h * DH:(h + 1) * DH]
                s = lax.dot_general(
                    q, k_, (((1,), (1,)), ((), ())),
                    preferred_element_type=F32,
                ) * 0.125
                p = jnp.exp(s)
                l = jnp.sum(p, axis=1, keepdims=True)
                o = jnp.dot(p.astype(BF), v, preferred_element_type=F32)
                osc[b, :, h * DH:(h + 1) * DH] = (o / l).astype(BF)
            y_ref[b] = jnp.dot(osc[b], wo, preferred_element_type=F32)

        rs_rdmas = []
        for k in (1, 2, 3):
            r = 3 - k
            tgt = lax.rem(my + k, N_DEV)
            ysend[k - 1] = y_ref[:, pl.ds(tgt * S_LOC, S_LOC), :].astype(BF)
            rdma = pltpu.make_async_remote_copy(
                src_ref=ysend.at[k - 1],
                dst_ref=rsbuf.at[r],
                send_sem=rs_send.at[k - 1],
                recv_sem=rs_recv.at[r],
                device_id=(tgt,),
                device_id_type=pl.DeviceIdType.MESH,
            )
            rdma.start()
            rs_rdmas.append(rdma)

        acc = y_ref[:, pl.ds(my * S_LOC, S_LOC), :]
        for r in (2, 0, 1):
            recv = pltpu.make_async_remote_copy(
                src_ref=ysend.at[0],
                dst_ref=rsbuf.at[r],
                send_sem=rs_send.at[0],
                recv_sem=rs_recv.at[r],
                device_id=(my,),
                device_id_type=pl.DeviceIdType.MESH,
            )
            recv.wait_recv()
            acc = acc + rsbuf[r].astype(F32)
        out_ref[...] = acc

        for rdma in ag_rdmas + rs_rdmas:
            rdma.wait_send()

    return pl.pallas_call(
        body,
        out_shape=jax.ShapeDtypeStruct((B, S_LOC, D), F32),
        in_specs=[pl.BlockSpec(memory_space=pltpu.VMEM)] * 5,
        out_specs=pl.BlockSpec(memory_space=pltpu.VMEM),
        scratch_shapes=[
            pltpu.VMEM((B, S_LOC, D), BF),
            pltpu.VMEM((3, B, S_LOC, D), BF),
            pltpu.VMEM((B, S, D), BF),
            pltpu.VMEM((B, S, D), BF),
            pltpu.VMEM((B, S, D), BF),
            pltpu.VMEM((B, S, D), BF),
            pltpu.VMEM((B, S, D), F32),
            pltpu.VMEM((3, B, S_LOC, D), BF),
            pltpu.VMEM((3, B, S_LOC, D), BF),
            pltpu.SemaphoreType.DMA((3,)),
            pltpu.SemaphoreType.DMA((3,)),
            pltpu.SemaphoreType.DMA((3,)),
            pltpu.SemaphoreType.DMA((3,)),
        ],
        compiler_params=pltpu.CompilerParams(collective_id=0),
    )(x, Wq, Wo, Wk, Wv)
